# baseline (speedup 1.0000x reference)
import functools

import jax
import jax.numpy as jnp
from jax import lax
from jax.experimental import pallas as pl
from jax.experimental.pallas import tpu as pltpu
from jax.experimental.pallas import tpu_sc as plsc

NC = 2
NS = 16
NW = NC * NS
CHUNK = 128
NBUF = 4


@functools.partial(jax.jit, static_argnums=(2, 3))
def _emb_lookup(weight, idx, n_chunks, d):
    mesh = plsc.VectorSubcoreMesh(core_axis_name="c", subcore_axis_name="s")
    b_total = NW * n_chunks * CHUNK

    @functools.partial(
        pl.kernel,
        mesh=mesh,
        out_type=jax.ShapeDtypeStruct((b_total, d), jnp.float32),
        compiler_params=pltpu.CompilerParams(use_tc_tiling_on_sc=False),
        scratch_types=[
            pltpu.VMEM((n_chunks, CHUNK), jnp.int32),
            pltpu.VMEM((NBUF, CHUNK, d), jnp.float32),
            pltpu.SemaphoreType.DMA((NBUF,)),
        ],
    )
    def body(table_hbm, idx_hbm, out_hbm, idx_v, rows_v, gsem):
        wid = lax.axis_index("s") * NC + lax.axis_index("c")
        base = wid * (n_chunks * CHUNK)
        pltpu.sync_copy(idx_hbm.at[wid], idx_v)

        def start_gather(chunk, buf):
            pltpu.async_copy(
                table_hbm.at[idx_v.at[chunk]], rows_v.at[buf], gsem.at[buf]
            )

        for b in range(NBUF):
            start_gather(b, b)

        @pl.loop(0, n_chunks, step=NBUF)
        def _(g):
            for b in range(NBUF):
                j = g + b
                pltpu.make_async_copy(
                    table_hbm.at[idx_v.at[j]], rows_v.at[b], gsem.at[b]
                ).wait()
                pltpu.sync_copy(
                    rows_v.at[b], out_hbm.at[pl.ds(base + j * CHUNK, CHUNK)]
                )

                @pl.when(j + NBUF < n_chunks)
                def _():
                    start_gather(j + NBUF, b)

    return body(weight, idx)


def kernel(input, weight):
    s0, s1 = input.shape
    v, d = weight.shape
    b_total = s0 * s1
    n_chunks = b_total // (NW * CHUNK)
    idx = input.reshape(NW, n_chunks, CHUNK).astype(jnp.int32)
    out = _emb_lookup(weight, idx, n_chunks, d)
    return out.reshape(s0, s1, d)
